# baseline (speedup 1.0000x reference)
import functools

import jax
import jax.numpy as jnp
from jax.experimental import pallas as pl

NUM_CODEBOOKS = 8
CODEBOOK_SIZE = 1024
COMMITMENT_WEIGHT = 0.25


def _rowsum256(x):
    y = x[:, :128] + x[:, 128:]
    p = y[:, 0:8]
    for i in range(1, 16):
        p = p + y[:, 8 * i:8 * (i + 1)]
    a = p[:, 0:4] + p[:, 4:8]
    b = a[:, 0:2] + a[:, 2:4]
    return b[:, 0:1] + b[:, 1:2]


def _esq_kernel(cb_ref, out_ref):
    for i in range(NUM_CODEBOOKS):
        W = cb_ref[i]
        e = _rowsum256(W * W)
        out_ref[i:i + 1, :] = jax.lax.transpose(e, (1, 0))


def _rvq_kernel(x_ref, cb_ref, esq_ref, q_ref, idx_ref, ssq_ref):
    x = x_ref[...]
    res = x
    q = jnp.zeros_like(x)
    ssq = jnp.zeros((), dtype=jnp.float32)
    iota = jax.lax.broadcasted_iota(jnp.int32, (1, CODEBOOK_SIZE), 1)
    for i in range(NUM_CODEBOOKS):
        W = cb_ref[i]
        hi = W.astype(jnp.bfloat16)
        r1 = W - hi.astype(jnp.float32)
        mid = r1.astype(jnp.bfloat16)
        lo = (r1 - mid.astype(jnp.float32)).astype(jnp.bfloat16)
        x_sq = _rowsum256(res * res)
        mm = jax.lax.dot_general(
            res, W, (((1,), (1,)), ((), ())),
            preferred_element_type=jnp.float32)
        d = (x_sq + esq_ref[i:i + 1, :]) - 2.0 * mm
        m = jnp.min(d, axis=1, keepdims=True)
        big = jnp.broadcast_to(jnp.int32(CODEBOOK_SIZE), d.shape)
        cand = jnp.where(d == m, jnp.broadcast_to(iota, d.shape), big)
        idx = jnp.min(cand, axis=1)
        idx_ref[i, :] = idx
        oh = (idx[:, None] == iota).astype(jnp.bfloat16)
        dn = (((1,), (0,)), ((), ()))
        zq = (jax.lax.dot_general(oh, hi, dn,
                                  preferred_element_type=jnp.float32)
              + jax.lax.dot_general(oh, mid, dn,
                                    preferred_element_type=jnp.float32)
              + jax.lax.dot_general(oh, lo, dn,
                                    preferred_element_type=jnp.float32))
        res = res - zq
        q = q + zq
        ssq = ssq + jnp.sum(res * res)
    q_ref[...] = q
    ssq2d = ssq[None, None]

    @pl.when(pl.program_id(0) == 0)
    def _init():
        ssq_ref[...] = ssq2d

    @pl.when(pl.program_id(0) != 0)
    def _acc():
        ssq_ref[...] = ssq_ref[...] + ssq2d


@functools.partial(jax.jit, static_argnames=())
def kernel(z, codebooks):
    Bz, Tz, D = z.shape
    N = Bz * Tz
    flat = z.reshape(N, D)
    TN = min(1024, N)
    grid = N // TN

    esq = pl.pallas_call(
        _esq_kernel,
        out_shape=jax.ShapeDtypeStruct((NUM_CODEBOOKS, CODEBOOK_SIZE),
                                       jnp.float32),
    )(codebooks)

    q, idx, ssq = pl.pallas_call(
        _rvq_kernel,
        grid=(grid,),
        in_specs=[
            pl.BlockSpec((TN, D), lambda n: (n, 0)),
            pl.BlockSpec((NUM_CODEBOOKS, CODEBOOK_SIZE, D),
                         lambda n: (0, 0, 0)),
            pl.BlockSpec((NUM_CODEBOOKS, CODEBOOK_SIZE), lambda n: (0, 0)),
        ],
        out_specs=[
            pl.BlockSpec((TN, D), lambda n: (n, 0)),
            pl.BlockSpec((NUM_CODEBOOKS, TN), lambda n: (0, n)),
            pl.BlockSpec((1, 1), lambda n: (0, 0)),
        ],
        out_shape=[
            jax.ShapeDtypeStruct((N, D), jnp.float32),
            jax.ShapeDtypeStruct((NUM_CODEBOOKS, N), jnp.int32),
            jax.ShapeDtypeStruct((1, 1), jnp.float32),
        ],
    )(flat, codebooks, esq)

    quantized = q.reshape(Bz, Tz, D)
    indices = jnp.transpose(idx).reshape(Bz, Tz, NUM_CODEBOOKS)
    vq_loss = (ssq[0, 0] * ((1.0 + COMMITMENT_WEIGHT) / (N * D))).astype(
        jnp.float32)
    return (quantized, indices, vq_loss)


# speedup vs baseline: 2.3220x; 2.3220x over previous
import functools

import jax
import jax.numpy as jnp
from jax.experimental import pallas as pl

NUM_CODEBOOKS = 8
CODEBOOK_SIZE = 1024
COMMITMENT_WEIGHT = 0.25


def _rowsum256_t(x):
    y = x[:, :128] + x[:, 128:]
    yt = jax.lax.transpose(y, (1, 0))
    p = yt[0:8]
    for i in range(1, 16):
        p = p + yt[8 * i:8 * (i + 1)]
    a = p[0:4] + p[4:8]
    b = a[0:2] + a[2:4]
    r = b[0:1] + b[1:2]
    return jax.lax.transpose(r, (1, 0))


def _prep_kernel(cb_ref, esq_ref, split_ref):
    for i in range(NUM_CODEBOOKS):
        W = cb_ref[i]
        e = _rowsum256_t(W * W)
        esq_ref[i:i + 1, :] = jax.lax.transpose(e, (1, 0))
        hi = W.astype(jnp.bfloat16)
        r1 = W - hi.astype(jnp.float32)
        mid = r1.astype(jnp.bfloat16)
        lo = (r1 - mid.astype(jnp.float32)).astype(jnp.bfloat16)
        split_ref[i, :, 0:256] = hi
        split_ref[i, :, 256:512] = mid
        split_ref[i, :, 512:768] = lo


def _rvq_kernel(x_ref, cb_ref, esq_ref, split_ref, q_ref, idx_ref, ssq_ref):
    x = x_ref[...]
    res = x
    q = jnp.zeros_like(x)
    ssq = jnp.zeros((), dtype=jnp.float32)
    iota = jax.lax.broadcasted_iota(jnp.int32, (1, CODEBOOK_SIZE), 1)
    for i in range(NUM_CODEBOOKS):
        W = cb_ref[i]
        x_sq = _rowsum256_t(res * res)
        mm = jax.lax.dot_general(
            res, W, (((1,), (1,)), ((), ())),
            preferred_element_type=jnp.float32)
        d = (x_sq + esq_ref[i:i + 1, :]) - 2.0 * mm
        m = jnp.min(d, axis=1, keepdims=True)
        big = jnp.broadcast_to(jnp.int32(CODEBOOK_SIZE), d.shape)
        cand = jnp.where(d == m, jnp.broadcast_to(iota, d.shape), big)
        idx = jnp.min(cand, axis=1)
        idx_ref[i, :] = idx
        oh = (idx[:, None] == iota).astype(jnp.bfloat16)
        parts = jax.lax.dot_general(
            oh, split_ref[i], (((1,), (0,)), ((), ())),
            preferred_element_type=jnp.float32)
        zq = (parts[:, 0:256] + parts[:, 256:512]) + parts[:, 512:768]
        res = res - zq
        q = q + zq
        ssq = ssq + jnp.sum(res * res)
    q_ref[...] = q
    ssq2d = ssq[None, None]

    @pl.when(pl.program_id(0) == 0)
    def _init():
        ssq_ref[...] = ssq2d

    @pl.when(pl.program_id(0) != 0)
    def _acc():
        ssq_ref[...] = ssq_ref[...] + ssq2d


@functools.partial(jax.jit, static_argnames=())
def kernel(z, codebooks):
    Bz, Tz, D = z.shape
    N = Bz * Tz
    flat = z.reshape(N, D)
    TN = min(1024, N)
    grid = N // TN

    esq, split = pl.pallas_call(
        _prep_kernel,
        out_shape=[
            jax.ShapeDtypeStruct((NUM_CODEBOOKS, CODEBOOK_SIZE),
                                 jnp.float32),
            jax.ShapeDtypeStruct((NUM_CODEBOOKS, CODEBOOK_SIZE, 3 * D),
                                 jnp.bfloat16),
        ],
    )(codebooks)

    q, idx, ssq = pl.pallas_call(
        _rvq_kernel,
        grid=(grid,),
        in_specs=[
            pl.BlockSpec((TN, D), lambda n: (n, 0)),
            pl.BlockSpec((NUM_CODEBOOKS, CODEBOOK_SIZE, D),
                         lambda n: (0, 0, 0)),
            pl.BlockSpec((NUM_CODEBOOKS, CODEBOOK_SIZE), lambda n: (0, 0)),
            pl.BlockSpec((NUM_CODEBOOKS, CODEBOOK_SIZE, 3 * D),
                         lambda n: (0, 0, 0)),
        ],
        out_specs=[
            pl.BlockSpec((TN, D), lambda n: (n, 0)),
            pl.BlockSpec((NUM_CODEBOOKS, TN), lambda n: (0, n)),
            pl.BlockSpec((1, 1), lambda n: (0, 0)),
        ],
        out_shape=[
            jax.ShapeDtypeStruct((N, D), jnp.float32),
            jax.ShapeDtypeStruct((NUM_CODEBOOKS, N), jnp.int32),
            jax.ShapeDtypeStruct((1, 1), jnp.float32),
        ],
    )(flat, codebooks, esq, split)

    quantized = q.reshape(Bz, Tz, D)
    indices = jnp.transpose(idx).reshape(Bz, Tz, NUM_CODEBOOKS)
    vq_loss = (ssq[0, 0] * ((1.0 + COMMITMENT_WEIGHT) / (N * D))).astype(
        jnp.float32)
    return (quantized, indices, vq_loss)
